# baseline (speedup 1.0000x reference)
import jax
import jax.numpy as jnp
from jax.experimental import pallas as pl
from jax.experimental.pallas import tpu as pltpu

_DIM = 1024
_N = 64
_H = 32
_T_CHUNK = 128
_B_BLK = 16
_EPS_NORM = 1e-6


def _consts():
    j2 = jax.lax.broadcasted_iota(jnp.int32, (2 * _N, 2 * _N), 0)
    o2 = jax.lax.broadcasted_iota(jnp.int32, (2 * _N, 2 * _N), 1)
    m2 = ((j2 // _N) == (o2 // _N)).astype(jnp.float32)
    i2 = jax.lax.broadcasted_iota(jnp.int32, (_H, _N), 0)
    c2 = jax.lax.broadcasted_iota(jnp.int32, (_H, _N), 1)
    maskp = ((c2 % _H) == i2).astype(jnp.float32)
    c3 = jax.lax.broadcasted_iota(jnp.int32, (_N, 2 * _N), 0)
    l3 = jax.lax.broadcasted_iota(jnp.int32, (_N, 2 * _N), 1)
    rp = ((c3 // _H) == (l3 // _N)).astype(jnp.float32)
    return m2, maskp, rp


def _cell_kernel(alpha_ref, x_ref, w_ref, s0_ref, out_ref, sf_ref,
                 kd_s, v_s, qd_s, md_s):
    t_idx = pl.program_id(1)
    alpha = alpha_ref[0]
    m2, maskp, rp = _consts()
    ones_n = jnp.ones((_N, _N), dtype=jnp.float32)

    xc = x_ref[...].reshape(_T_CHUNK * _B_BLK, _DIM)
    proj = jnp.dot(xc, w_ref[...], preferred_element_type=jnp.float32)
    k = proj[:, :_N]
    v = proj[:, _N:2 * _N]
    q = proj[:, 2 * _N:3 * _N]
    m = proj[:, 3 * _N:]
    kk_bc = jnp.dot(k * k, ones_n, preferred_element_type=jnp.float32)
    mm_bc = jnp.dot(m * m, ones_n, preferred_element_type=jnp.float32)
    k = k * (1.0 / (jnp.sqrt(kk_bc) + _EPS_NORM))
    m = m * (1.0 / (jnp.sqrt(mm_bc) + _EPS_NORM))
    kd_s[...] = jnp.concatenate([k, k], axis=-1).reshape(_T_CHUNK, _B_BLK,
                                                         2 * _N)
    md_s[...] = jnp.concatenate([m, m], axis=-1).reshape(_T_CHUNK, _B_BLK,
                                                         2 * _N)
    qd_s[...] = jnp.concatenate([q, q], axis=-1).reshape(_T_CHUNK, _B_BLK,
                                                         2 * _N)
    v_s[...] = v.reshape(_T_CHUNK, _B_BLK, _N)

    @pl.when(t_idx == 0)
    def _():
        sf_ref[...] = s0_ref[...]

    S0u = sf_ref[...]
    Sp0 = jnp.concatenate([S0u[:, :_H, :], S0u[:, _H:, :]], axis=-1)

    def step(t, S):
        kd = kd_s[t][:, None, :]
        qd = qd_s[t][:, None, :]
        md = md_s[t][:, None, :]
        PP = jnp.concatenate([(S * md).reshape(_B_BLK * _H, 2 * _N),
                              (S * kd).reshape(_B_BLK * _H, 2 * _N)], axis=0)
        DD = jnp.dot(PP, m2, preferred_element_type=jnp.float32)
        Sm_bc = DD[:_B_BLK * _H].reshape(_B_BLK, _H, 2 * _N)
        Sk_bc = DD[_B_BLK * _H:].reshape(_B_BLK, _H, 2 * _N)
        vrow = (jnp.broadcast_to(v_s[t][:, None, :], (_B_BLK, _H, _N))
                * maskp[None]).reshape(_B_BLK * _H, _N)
        vb = jnp.dot(vrow, rp, preferred_element_type=jnp.float32
                     ).reshape(_B_BLK, _H, 2 * _N)
        G = jax.nn.sigmoid(Sm_bc * kd + alpha * S)
        S_new = G * S + (vb - Sk_bc) * kd
        Pq = S_new * qd
        Sq_0 = jnp.sum(Pq[:, :, :_N], axis=-1)
        Sq_1 = jnp.sum(Pq[:, :, _N:], axis=-1)
        out_ref[pl.ds(t, 1), :, :] = jnp.concatenate([Sq_0, Sq_1],
                                                     axis=-1)[None]
        return S_new

    def body(t8, S):
        for u in range(8):
            S = step(8 * t8 + u, S)
        return S

    S_fin = jax.lax.fori_loop(0, _T_CHUNK // 8, body, Sp0)
    sf_ref[...] = jnp.concatenate([S_fin[:, :, :_N], S_fin[:, :, _N:]],
                                  axis=1)
    Sq_all = out_ref[...]
    out_ref[...] = Sq_all * Sq_all * jax.nn.sigmoid(Sq_all)


@jax.jit
def kernel(x, S0, W_kvqm, alpha):
    T, B, D = x.shape
    n = W_kvqm.shape[0] // 4
    wt = W_kvqm.T
    alpha_arr = jnp.reshape(alpha, (1,)).astype(jnp.float32)
    grid = (B // _B_BLK, T // _T_CHUNK)
    out, s_fin = pl.pallas_call(
        _cell_kernel,
        grid=grid,
        in_specs=[
            pl.BlockSpec(memory_space=pltpu.SMEM),
            pl.BlockSpec((_T_CHUNK, _B_BLK, D), lambda b, t: (t, b, 0)),
            pl.BlockSpec((D, 4 * _N), lambda b, t: (0, 0)),
            pl.BlockSpec((_B_BLK, _N, _N), lambda b, t: (b, 0, 0)),
        ],
        out_specs=[
            pl.BlockSpec((_T_CHUNK, _B_BLK, _N), lambda b, t: (t, b, 0)),
            pl.BlockSpec((_B_BLK, _N, _N), lambda b, t: (b, 0, 0)),
        ],
        out_shape=[
            jax.ShapeDtypeStruct((T, B, n), jnp.float32),
            jax.ShapeDtypeStruct((B, n, n), jnp.float32),
        ],
        scratch_shapes=[
            pltpu.VMEM((_T_CHUNK, _B_BLK, 2 * _N), jnp.float32),
            pltpu.VMEM((_T_CHUNK, _B_BLK, _N), jnp.float32),
            pltpu.VMEM((_T_CHUNK, _B_BLK, 2 * _N), jnp.float32),
            pltpu.VMEM((_T_CHUNK, _B_BLK, 2 * _N), jnp.float32),
        ],
        compiler_params=pltpu.CompilerParams(
            dimension_semantics=("parallel", "arbitrary"),
        ),
    )(alpha_arr, x, wt, S0)
    return out, s_fin


# speedup vs baseline: 1.9686x; 1.9686x over previous
import jax
import jax.numpy as jnp
from jax.experimental import pallas as pl
from jax.experimental.pallas import tpu as pltpu

_DIM = 1024
_N = 64
_T_CHUNK = 64
_B_BLK = 16
_EPS_NORM = 1e-6


def _cell_kernel(alpha_ref, x_ref, w_ref, s0_ref, out_ref, sf_ref,
                 k_s, v_s, q_s, m_s):
    t_idx = pl.program_id(1)
    alpha = alpha_ref[0]
    ones_n = jnp.ones((_N, _N), dtype=jnp.float32)

    xc = x_ref[...].reshape(_T_CHUNK * _B_BLK, _DIM)
    proj = jnp.dot(xc, w_ref[...], preferred_element_type=jnp.float32)
    k = proj[:, :_N]
    v = proj[:, _N:2 * _N]
    q = proj[:, 2 * _N:3 * _N]
    m = proj[:, 3 * _N:]
    kk_bc = jnp.dot(k * k, ones_n, preferred_element_type=jnp.float32)
    mm_bc = jnp.dot(m * m, ones_n, preferred_element_type=jnp.float32)
    k = k * (1.0 / (jnp.sqrt(kk_bc) + _EPS_NORM))
    m = m * (1.0 / (jnp.sqrt(mm_bc) + _EPS_NORM))
    k_s[...] = k.reshape(_T_CHUNK, _B_BLK, _N)
    v_s[...] = v.reshape(_T_CHUNK, _B_BLK, _N)
    q_s[...] = q.reshape(_T_CHUNK, _B_BLK, _N)
    m_s[...] = m.reshape(_T_CHUNK, _B_BLK, _N)

    @pl.when(t_idx == 0)
    def _():
        sf_ref[...] = s0_ref[...]

    def step(t, S):
        k = k_s[t][:, None, :]
        q = q_s[t][:, None, :]
        m = m_s[t][:, None, :]
        v = v_s[t][:, :, None]
        PP = jnp.concatenate([(S * m).reshape(_B_BLK * _N, _N),
                              (S * k).reshape(_B_BLK * _N, _N)], axis=0)
        DD = jnp.dot(PP, ones_n, preferred_element_type=jnp.float32)
        Sm_bc = DD[:_B_BLK * _N].reshape(_B_BLK, _N, _N)
        Sk_bc = DD[_B_BLK * _N:].reshape(_B_BLK, _N, _N)
        G = jax.nn.sigmoid(Sm_bc * k + alpha * S)
        S_new = G * S + (v - Sk_bc) * k
        Sq = jnp.sum(S_new * q, axis=-1)
        out_ref[pl.ds(t, 1), :, :] = Sq[None]
        return S_new

    def body(t32, S):
        for u in range(32):
            S = step(32 * t32 + u, S)
        return S

    S_fin = jax.lax.fori_loop(0, _T_CHUNK // 32, body, sf_ref[...])
    sf_ref[...] = S_fin
    Sq_all = out_ref[...]
    out_ref[...] = Sq_all * Sq_all * jax.nn.sigmoid(Sq_all)


@jax.jit
def kernel(x, S0, W_kvqm, alpha):
    T, B, D = x.shape
    n = W_kvqm.shape[0] // 4
    wt = W_kvqm.T
    alpha_arr = jnp.reshape(alpha, (1,)).astype(jnp.float32)
    grid = (B // _B_BLK, T // _T_CHUNK)
    out, s_fin = pl.pallas_call(
        _cell_kernel,
        grid=grid,
        in_specs=[
            pl.BlockSpec(memory_space=pltpu.SMEM),
            pl.BlockSpec((_T_CHUNK, _B_BLK, D), lambda b, t: (t, b, 0)),
            pl.BlockSpec((D, 4 * _N), lambda b, t: (0, 0)),
            pl.BlockSpec((_B_BLK, _N, _N), lambda b, t: (b, 0, 0)),
        ],
        out_specs=[
            pl.BlockSpec((_T_CHUNK, _B_BLK, _N), lambda b, t: (t, b, 0)),
            pl.BlockSpec((_B_BLK, _N, _N), lambda b, t: (b, 0, 0)),
        ],
        out_shape=[
            jax.ShapeDtypeStruct((T, B, n), jnp.float32),
            jax.ShapeDtypeStruct((B, n, n), jnp.float32),
        ],
        scratch_shapes=[pltpu.VMEM((_T_CHUNK, _B_BLK, _N), jnp.float32)
                        for _ in range(4)],
        compiler_params=pltpu.CompilerParams(
            dimension_semantics=("core_parallel", "arbitrary"),
        ),
    )(alpha_arr, x, wt, S0)
    return out, s_fin


# speedup vs baseline: 1.9933x; 1.0125x over previous
import jax
import jax.numpy as jnp
from jax.experimental import pallas as pl
from jax.experimental.pallas import tpu as pltpu

_DIM = 1024
_N = 64
_T_CHUNK = 64
_B_BLK = 16
_EPS_NORM = 1e-6


def _cell_kernel(alpha_ref, x_ref, w_ref, s0_ref, out_ref, sf_ref,
                 k_s, v_s, q_s, m_s):
    t_idx = pl.program_id(1)
    alpha = alpha_ref[0]
    ones_n = jnp.ones((_N, _N), dtype=jnp.float32)

    xc = x_ref[...].reshape(_T_CHUNK * _B_BLK, _DIM)
    proj = jnp.dot(xc, w_ref[...], preferred_element_type=jnp.float32)
    k = proj[:, :_N]
    v = proj[:, _N:2 * _N]
    q = proj[:, 2 * _N:3 * _N]
    m = proj[:, 3 * _N:]
    kk_bc = jnp.dot(k * k, ones_n, preferred_element_type=jnp.float32)
    mm_bc = jnp.dot(m * m, ones_n, preferred_element_type=jnp.float32)
    k = k * (1.0 / (jnp.sqrt(kk_bc) + _EPS_NORM))
    m = m * (1.0 / (jnp.sqrt(mm_bc) + _EPS_NORM))
    k_s[...] = k.reshape(_T_CHUNK, _B_BLK, _N)
    v_s[...] = v.reshape(_T_CHUNK, _B_BLK, _N)
    q_s[...] = q.reshape(_T_CHUNK, _B_BLK, _N)
    m_s[...] = m.reshape(_T_CHUNK, _B_BLK, _N)

    @pl.when(t_idx == 0)
    def _():
        sf_ref[...] = s0_ref[...]

    def step(t, S):
        k = k_s[t][:, None, :]
        q = q_s[t][:, None, :]
        m = m_s[t][:, None, :]
        v = v_s[t][:, :, None]
        PP = jnp.concatenate([(S * m).reshape(_B_BLK * _N, _N),
                              (S * k).reshape(_B_BLK * _N, _N)], axis=0)
        DD = jnp.dot(PP, ones_n, preferred_element_type=jnp.float32)
        Sm_bc = DD[:_B_BLK * _N].reshape(_B_BLK, _N, _N)
        Sk_bc = DD[_B_BLK * _N:].reshape(_B_BLK, _N, _N)
        G = jax.nn.sigmoid(Sm_bc * k + alpha * S)
        S_new = G * S + (v - Sk_bc) * k
        Sq = jnp.sum(S_new * q, axis=-1)
        out_ref[pl.ds(t, 1), :, :] = Sq[None]
        return S_new

    def body(t64, S):
        for u in range(64):
            S = step(64 * t64 + u, S)
        return S

    S_fin = jax.lax.fori_loop(0, _T_CHUNK // 64, body, sf_ref[...])
    sf_ref[...] = S_fin
    Sq_all = out_ref[...]
    out_ref[...] = Sq_all * Sq_all * jax.nn.sigmoid(Sq_all)


@jax.jit
def kernel(x, S0, W_kvqm, alpha):
    T, B, D = x.shape
    n = W_kvqm.shape[0] // 4
    wt = W_kvqm.T
    alpha_arr = jnp.reshape(alpha, (1,)).astype(jnp.float32)
    grid = (B // _B_BLK, T // _T_CHUNK)
    out, s_fin = pl.pallas_call(
        _cell_kernel,
        grid=grid,
        in_specs=[
            pl.BlockSpec(memory_space=pltpu.SMEM),
            pl.BlockSpec((_T_CHUNK, _B_BLK, D), lambda b, t: (t, b, 0)),
            pl.BlockSpec((D, 4 * _N), lambda b, t: (0, 0)),
            pl.BlockSpec((_B_BLK, _N, _N), lambda b, t: (b, 0, 0)),
        ],
        out_specs=[
            pl.BlockSpec((_T_CHUNK, _B_BLK, _N), lambda b, t: (t, b, 0)),
            pl.BlockSpec((_B_BLK, _N, _N), lambda b, t: (b, 0, 0)),
        ],
        out_shape=[
            jax.ShapeDtypeStruct((T, B, n), jnp.float32),
            jax.ShapeDtypeStruct((B, n, n), jnp.float32),
        ],
        scratch_shapes=[pltpu.VMEM((_T_CHUNK, _B_BLK, _N), jnp.float32)
                        for _ in range(4)],
        compiler_params=pltpu.CompilerParams(
            dimension_semantics=("core_parallel", "arbitrary"),
        ),
    )(alpha_arr, x, wt, S0)
    return out, s_fin
